# speedup vs baseline: 1.2909x; 1.2909x over previous
import jax
import jax.numpy as jnp
from jax import lax
from jax.experimental import pallas as pl
from jax.experimental.pallas import tpu as pltpu

N_DIM = 128
HIDDEN = 256
K = 8
EPS = 1e-11


def _recon_kernel(fcap_ref, sc2_ref, cls_ref, wt_ref, b_ref, y1h_ref,
                  mask_ref, out_ref):
    f32 = jnp.float32

    sc2 = sc2_ref[0]
    sq = jnp.sum(sc2 * sc2, axis=1, keepdims=True)
    scale = sq / (1.0 + sq)
    sc2n = sc2 / jnp.sqrt(sq + EPS)

    iota_n = lax.broadcasted_iota(jnp.int32, (64, 1), 0)
    row_ids = lax.broadcasted_iota(jnp.int32, (K, 64), 0)

    def topk_body(k, carry):
        onehot, msk = carry
        m = jnp.max(msk)
        eq = msk == m
        idx = jnp.min(jnp.where(eq, iota_n, 64))
        sel = iota_n == idx
        onehot = jnp.where((row_ids == k) & sel.T, 1.0, onehot)
        msk = jnp.where(sel, -1.0, msk)
        return onehot, msk

    onehot, _ = lax.fori_loop(
        0, K, topk_body, (jnp.zeros((K, 64), f32), scale))

    top_scales = lax.dot_general(onehot, scale, (((1,), (0,)), ((), ())),
                                 preferred_element_type=f32)
    sel_rows = lax.dot_general(onehot, sc2n, (((1,), (0,)), ((), ())),
                               preferred_element_type=f32)

    cc_raw = lax.dot_general(y1h_ref[0], cls_ref[0], (((1,), (0,)), ((), ())),
                             preferred_element_type=f32)
    sqc = jnp.sum(cc_raw * cc_raw, axis=1, keepdims=True)
    ccn = cc_raw / jnp.sqrt(sqc + EPS)
    scale1 = sqc / (1.0 + sqc)

    wt = wt_ref[...]
    bias = b_ref[...]
    s8 = jnp.maximum(
        lax.dot_general(sel_rows, wt, (((1,), (0,)), ((), ())),
                        preferred_element_type=f32) + bias, 0.0)
    s1 = jnp.maximum(
        lax.dot_general(ccn, wt, (((1,), (0,)), ((), ())),
                        preferred_element_type=f32) + bias, 0.0)

    W = jnp.sum(top_scales) + scale1
    v = (lax.dot_general(top_scales, s8, (((0,), (0,)), ((), ())),
                         preferred_element_type=f32)
         + scale1 * s1)
    c = (jnp.sum(top_scales * jnp.sum(s8 * s8, axis=1, keepdims=True))
         + scale1 * jnp.sum(s1 * s1, axis=1, keepdims=True))

    F = jnp.maximum(fcap_ref[0], 0.0)
    G = lax.dot_general(F, F, (((1,), (1,)), ((), ())),
                        preferred_element_type=f32)
    u_col = lax.dot_general(F, v, (((1,), (1,)), ((), ())),
                            preferred_element_type=f32)
    u_row = lax.dot_general(v, F, (((1,), (1,)), ((), ())),
                            preferred_element_type=f32)

    adj = W * G + u_col + u_row + c
    out_ref[0] = adj * mask_ref[0]


def kernel(first_capsule, second_capsule, class_capsule, fc1_w, fc1_b, y,
           first_capsule_mask):
    B, M, H = first_capsule.shape
    f32 = jnp.float32

    y_onehot = jax.nn.one_hot(y, class_capsule.shape[1], dtype=f32)
    y_onehot = y_onehot.reshape(B, 1, class_capsule.shape[1])
    mask3 = first_capsule_mask.astype(f32).reshape(B, M, 1)
    fc1_wt = fc1_w.T
    fc1_b2 = fc1_b.reshape(1, H)

    out = pl.pallas_call(
        _recon_kernel,
        grid=(B,),
        in_specs=[
            pl.BlockSpec((1, M, H), lambda b: (b, 0, 0)),
            pl.BlockSpec((1,) + second_capsule.shape[1:], lambda b: (b, 0, 0)),
            pl.BlockSpec((1,) + class_capsule.shape[1:], lambda b: (b, 0, 0)),
            pl.BlockSpec(fc1_wt.shape, lambda b: (0, 0)),
            pl.BlockSpec(fc1_b2.shape, lambda b: (0, 0)),
            pl.BlockSpec((1, 1, class_capsule.shape[1]), lambda b: (b, 0, 0)),
            pl.BlockSpec((1, M, 1), lambda b: (b, 0, 0)),
        ],
        out_specs=pl.BlockSpec((1, M, M), lambda b: (b, 0, 0)),
        out_shape=jax.ShapeDtypeStruct((B, M, M), f32),
        compiler_params=pltpu.CompilerParams(
            dimension_semantics=("arbitrary",)),
    )(first_capsule, second_capsule, class_capsule, fc1_wt, fc1_b2,
      y_onehot, mask3)
    return out


# speedup vs baseline: 1.8735x; 1.4512x over previous
import jax
import jax.numpy as jnp
from jax import lax
from jax.experimental import pallas as pl
from jax.experimental.pallas import tpu as pltpu

N_DIM = 128
HIDDEN = 256
K = 8
EPS = 1e-11


def _select_kernel(sc2_ref, cls_ref, y1h_ref, wt_ref, b_ref, v_ref, s_ref):
    f32 = jnp.float32
    B = sc2_ref.shape[0]

    sc2 = sc2_ref[...]
    sq = jnp.sum(sc2 * sc2, axis=2, keepdims=True)
    scale = sq[..., 0] / (1.0 + sq[..., 0])
    sc2n = sc2 / jnp.sqrt(sq + EPS)

    iota_n = lax.broadcasted_iota(jnp.int32, (B, 64), 1)

    rows = []
    ws = []
    msk = scale
    for _ in range(K):
        m = jnp.max(msk, axis=1, keepdims=True)
        idx = jnp.min(jnp.where(msk == m, iota_n, 64), axis=1, keepdims=True)
        sel = (iota_n == idx).astype(f32)
        rows.append(jnp.sum(sel[:, :, None] * sc2n, axis=1))
        ws.append(m)
        msk = jnp.where(sel > 0, -1.0, msk)

    y1h = y1h_ref[...][:, 0, :]
    cc_raw = jnp.sum(y1h[:, :, None] * cls_ref[...], axis=1)
    sqc = jnp.sum(cc_raw * cc_raw, axis=1, keepdims=True)
    rows.append(cc_raw / jnp.sqrt(sqc + EPS))
    ws.append(sqc / (1.0 + sqc))

    S = jnp.concatenate(rows, axis=0)
    w_all = jnp.concatenate(ws, axis=0)

    sproc = jnp.maximum(
        lax.dot_general(S, wt_ref[...], (((1,), (0,)), ((), ())),
                        preferred_element_type=f32) + b_ref[...], 0.0)

    s3 = jnp.reshape(sproc, (K + 1, B, HIDDEN))
    w3 = jnp.reshape(w_all, (K + 1, B, 1))

    v = jnp.sum(w3 * s3, axis=0)
    Wt = jnp.sum(w3[..., 0], axis=0, keepdims=True).T
    rs = jnp.sum(s3 * s3, axis=2, keepdims=True)
    c = jnp.sum(w3 * rs, axis=0)

    v_ref[...] = jnp.reshape(v, (B, 1, HIDDEN))
    pad = jnp.zeros((B, 126), f32)
    s_ref[...] = jnp.reshape(
        jnp.concatenate([Wt, c, pad], axis=1), (B, 1, 128))


def _dense_kernel(fcap_ref, v_ref, s_ref, mask_ref, out_ref):
    f32 = jnp.float32
    F = jnp.maximum(fcap_ref[0], 0.0)
    v = v_ref[0]
    W = s_ref[0][0:1, 0:1]
    c = s_ref[0][0:1, 1:2]

    G = lax.dot_general(F, F, (((1,), (1,)), ((), ())),
                        preferred_element_type=f32)
    u_col = lax.dot_general(F, v, (((1,), (1,)), ((), ())),
                            preferred_element_type=f32)
    u_row = lax.dot_general(v, F, (((1,), (1,)), ((), ())),
                            preferred_element_type=f32)

    adj = W * G + u_col + u_row + c
    out_ref[0] = adj * mask_ref[0]


def kernel(first_capsule, second_capsule, class_capsule, fc1_w, fc1_b, y,
           first_capsule_mask):
    B, M, H = first_capsule.shape
    NC = class_capsule.shape[1]
    f32 = jnp.float32

    y_onehot = jax.nn.one_hot(y, NC, dtype=f32).reshape(B, 1, NC)
    mask3 = first_capsule_mask.astype(f32).reshape(B, M, 1)
    fc1_wt = fc1_w.T
    fc1_b2 = fc1_b.reshape(1, H)

    v_all, s_all = pl.pallas_call(
        _select_kernel,
        in_specs=[
            pl.BlockSpec(second_capsule.shape, lambda: (0, 0, 0)),
            pl.BlockSpec(class_capsule.shape, lambda: (0, 0, 0)),
            pl.BlockSpec((B, 1, NC), lambda: (0, 0, 0)),
            pl.BlockSpec(fc1_wt.shape, lambda: (0, 0)),
            pl.BlockSpec(fc1_b2.shape, lambda: (0, 0)),
        ],
        out_specs=[
            pl.BlockSpec((B, 1, H), lambda: (0, 0, 0)),
            pl.BlockSpec((B, 1, 128), lambda: (0, 0, 0)),
        ],
        out_shape=[
            jax.ShapeDtypeStruct((B, 1, H), f32),
            jax.ShapeDtypeStruct((B, 1, 128), f32),
        ],
    )(second_capsule, class_capsule, y_onehot, fc1_wt, fc1_b2)

    out = pl.pallas_call(
        _dense_kernel,
        grid=(B,),
        in_specs=[
            pl.BlockSpec((1, M, H), lambda b: (b, 0, 0)),
            pl.BlockSpec((1, 1, H), lambda b: (b, 0, 0)),
            pl.BlockSpec((1, 1, 128), lambda b: (b, 0, 0)),
            pl.BlockSpec((1, M, 1), lambda b: (b, 0, 0)),
        ],
        out_specs=pl.BlockSpec((1, M, M), lambda b: (b, 0, 0)),
        out_shape=jax.ShapeDtypeStruct((B, M, M), f32),
        compiler_params=pltpu.CompilerParams(
            dimension_semantics=("arbitrary",)),
    )(first_capsule, v_all, s_all, mask3)
    return out


# speedup vs baseline: 1.9599x; 1.0462x over previous
import jax
import jax.numpy as jnp
from jax import lax
from jax.experimental import pallas as pl
from jax.experimental.pallas import tpu as pltpu

N_DIM = 128
HIDDEN = 256
K = 8
EPS = 1e-11


def _select_kernel(sc2_ref, cls_ref, y_ref, w_ref, b_ref, v_ref, s_ref):
    f32 = jnp.float32
    B, N2, D = sc2_ref.shape
    NC = cls_ref.shape[1]

    sc2 = sc2_ref[...]
    ones_d = jnp.ones((D, 1), f32)
    sq = lax.dot_general(jnp.reshape(sc2 * sc2, (B * N2, D)), ones_d,
                         (((1,), (0,)), ((), ())),
                         preferred_element_type=f32)
    sq3 = jnp.reshape(sq, (B, N2, 1))
    scale3 = sq3 / (1.0 + sq3)
    sc2n = sc2 / jnp.sqrt(sq3 + EPS)

    iota_n = lax.broadcasted_iota(jnp.int32, (B, N2, 1), 1)
    msk = scale3
    keep = jnp.zeros((B, N2, 1), f32)
    for _ in range(K):
        m = jnp.max(msk, axis=1, keepdims=True)
        idx = jnp.min(jnp.where(msk == m, iota_n, N2), axis=1, keepdims=True)
        sel = iota_n == idx
        keep = jnp.where(sel, 1.0, keep)
        msk = jnp.where(sel, -1.0, msk)
    wsel = scale3 * keep

    iota_c = lax.broadcasted_iota(jnp.int32, (B, NC, 1), 1)
    y3 = jnp.reshape(y_ref[...], (B, 1, 1))
    y1h = (iota_c == y3).astype(f32)
    cc_raw = jnp.sum(y1h * cls_ref[...], axis=1)
    sqc = lax.dot_general(cc_raw * cc_raw, ones_d, (((1,), (0,)), ((), ())),
                          preferred_element_type=f32)
    ccn = cc_raw / jnp.sqrt(sqc + EPS)
    scale1 = sqc / (1.0 + sqc)

    fw = w_ref[...]
    bias = b_ref[...]
    sproc = jnp.maximum(
        lax.dot_general(jnp.reshape(sc2n, (B * N2, D)), fw,
                        (((1,), (1,)), ((), ())),
                        preferred_element_type=f32) + bias, 0.0)
    s1 = jnp.maximum(
        lax.dot_general(ccn, fw, (((1,), (1,)), ((), ())),
                        preferred_element_type=f32) + bias, 0.0)

    ones_h = jnp.ones((HIDDEN, 1), f32)
    rowsq = lax.dot_general(sproc * sproc, ones_h, (((1,), (0,)), ((), ())),
                            preferred_element_type=f32)
    s1sq = lax.dot_general(s1 * s1, ones_h, (((1,), (0,)), ((), ())),
                           preferred_element_type=f32)

    sproc3 = jnp.reshape(sproc, (B, N2, HIDDEN))
    rowsq3 = jnp.reshape(rowsq, (B, N2, 1))

    v = jnp.sum(wsel * sproc3, axis=1) + scale1 * s1
    Wt = jnp.sum(wsel, axis=1) + scale1
    c = jnp.sum(wsel * rowsq3, axis=1) + scale1 * s1sq

    v_ref[...] = jnp.reshape(v, (B, 1, HIDDEN))
    pad = jnp.zeros((B, 126), f32)
    s_ref[...] = jnp.reshape(
        jnp.concatenate([Wt, c, pad], axis=1), (B, 1, 128))


def _dense_kernel(fcap_ref, v_ref, s_ref, mask_ref, out_ref):
    f32 = jnp.float32
    F = jnp.maximum(fcap_ref[0], 0.0)
    v = v_ref[0]
    W = s_ref[0][0:1, 0:1]
    c = s_ref[0][0:1, 1:2]

    G = lax.dot_general(F, F, (((1,), (1,)), ((), ())),
                        preferred_element_type=f32)
    u_col = lax.dot_general(F, v, (((1,), (1,)), ((), ())),
                            preferred_element_type=f32)
    u_row = lax.dot_general(v, F, (((1,), (1,)), ((), ())),
                            preferred_element_type=f32)

    adj = W * G + u_col + u_row + c
    out_ref[0] = adj * mask_ref[0]


def kernel(first_capsule, second_capsule, class_capsule, fc1_w, fc1_b, y,
           first_capsule_mask):
    B, M, H = first_capsule.shape
    NC = class_capsule.shape[1]
    f32 = jnp.float32

    y2 = y.astype(jnp.int32).reshape(B, 1)
    mask3 = first_capsule_mask.astype(f32).reshape(B, M, 1)
    fc1_b2 = fc1_b.reshape(1, H)

    v_all, s_all = pl.pallas_call(
        _select_kernel,
        in_specs=[
            pl.BlockSpec(second_capsule.shape, lambda: (0, 0, 0)),
            pl.BlockSpec(class_capsule.shape, lambda: (0, 0, 0)),
            pl.BlockSpec((B, 1), lambda: (0, 0)),
            pl.BlockSpec(fc1_w.shape, lambda: (0, 0)),
            pl.BlockSpec(fc1_b2.shape, lambda: (0, 0)),
        ],
        out_specs=[
            pl.BlockSpec((B, 1, H), lambda: (0, 0, 0)),
            pl.BlockSpec((B, 1, 128), lambda: (0, 0, 0)),
        ],
        out_shape=[
            jax.ShapeDtypeStruct((B, 1, H), f32),
            jax.ShapeDtypeStruct((B, 1, 128), f32),
        ],
    )(second_capsule, class_capsule, y2, fc1_w, fc1_b2)

    out = pl.pallas_call(
        _dense_kernel,
        grid=(B,),
        in_specs=[
            pl.BlockSpec((1, M, H), lambda b: (b, 0, 0)),
            pl.BlockSpec((1, 1, H), lambda b: (b, 0, 0)),
            pl.BlockSpec((1, 1, 128), lambda b: (b, 0, 0)),
            pl.BlockSpec((1, M, 1), lambda b: (b, 0, 0)),
        ],
        out_specs=pl.BlockSpec((1, M, M), lambda b: (b, 0, 0)),
        out_shape=jax.ShapeDtypeStruct((B, M, M), f32),
        compiler_params=pltpu.CompilerParams(
            dimension_semantics=("arbitrary",)),
    )(first_capsule, v_all, s_all, mask3)
    return out
